# baseline (speedup 1.0000x reference)
import functools

import jax
import jax.numpy as jnp
from jax import lax
from jax.experimental import pallas as pl
from jax.experimental.pallas import tpu as pltpu
from jax.experimental.pallas import tpu_sc as plsc

EMB = 64
CH = 128


def _sc_gather(table, idx_t, idx_q, n_t, n_q, nw):
    info = plsc.get_sparse_core_info()
    nc, ns = info.num_cores, info.num_subcores
    assert nc * ns == nw
    kt = idx_t.shape[1]
    kq = idx_q.shape[1]
    per_t = kt * CH
    per_q = kq * CH

    mesh = plsc.VectorSubcoreMesh(core_axis_name="c", subcore_axis_name="s")

    @functools.partial(
        pl.kernel,
        out_type=(
            jax.ShapeDtypeStruct((n_t, EMB), jnp.float32),
            jax.ShapeDtypeStruct((n_q, EMB), jnp.float32),
        ),
        mesh=mesh,
        compiler_params=pltpu.CompilerParams(use_tc_tiling_on_sc=False),
        scratch_types=[
            pltpu.VMEM((kt, CH), jnp.int32),
            pltpu.VMEM((kq, CH), jnp.int32),
            pltpu.VMEM((CH, EMB), jnp.float32),
            pltpu.VMEM((CH, EMB), jnp.float32),
            pltpu.SemaphoreType.DMA,
            pltpu.SemaphoreType.DMA,
            pltpu.SemaphoreType.DMA,
            pltpu.SemaphoreType.DMA,
        ],
    )
    def k(table_h, idxt_h, idxq_h, out_t_h, out_q_h,
          idxt_v, idxq_v, rows0, rows1, gsem0, gsem1, ssem0, ssem1):
        wid = lax.axis_index("s") * nc + lax.axis_index("c")
        pltpu.sync_copy(idxt_h.at[wid], idxt_v)
        pltpu.sync_copy(idxq_h.at[wid], idxq_v)

        def run(idx_v, out_h, base, kk):
            assert kk >= 2
            rows = (rows0, rows1)
            gsem = (gsem0, gsem1)
            ssem = (ssem0, ssem1)
            pltpu.async_copy(table_h.at[idx_v.at[0]], rows[0], gsem[0])

            def step(jj, _):
                for b in range(2):
                    @pl.when(jj % 2 == b)
                    def _():
                        @pl.when(jj + 1 < kk)
                        def _():
                            @pl.when(jj >= 1)
                            def _():
                                pltpu.make_async_copy(
                                    rows[1 - b], out_h.at[pl.ds(base, CH)],
                                    ssem[1 - b]).wait()

                            pltpu.async_copy(
                                table_h.at[idx_v.at[jj + 1]], rows[1 - b],
                                gsem[1 - b])

                        pltpu.make_async_copy(
                            table_h.at[idx_v.at[jj]], rows[b], gsem[b]).wait()
                        pltpu.async_copy(
                            rows[b], out_h.at[pl.ds(base + jj * CH, CH)],
                            ssem[b])
                return 0

            lax.fori_loop(0, kk, step, 0)
            for jj in (kk - 2, kk - 1):
                pltpu.make_async_copy(
                    rows[jj % 2], out_h.at[pl.ds(base, CH)],
                    ssem[jj % 2]).wait()

        run(idxt_v, out_t_h, wid * per_t, kt)
        run(idxq_v, out_q_h, wid * per_q, kq)

    return k(table, idx_t, idx_q)


def _tc_body(rt_ref, rq_ref, wi_ref, bi_ref, wq_ref, bq_ref,
             w1_ref, b1_ref, w2_ref, b2_ref, out_ref, *, s):
    t = jnp.tanh(rt_ref[...] @ wi_ref[...] + bi_ref[...])
    tm = jnp.mean(t.reshape(s, 1000, EMB), axis=1)
    q = jnp.tanh(rq_ref[...] @ wq_ref[...] + bq_ref[...])
    qm = jnp.mean(q.reshape(s, 100, EMB), axis=1)
    pool = jnp.concatenate([tm, qm], axis=-1)
    h = jnp.maximum(pool @ w1_ref[...] + b1_ref[...], 0.0)
    out_ref[...] = h @ w2_ref[...] + b2_ref[...]


def kernel(input_x, input_x_i, input_x_q, table,
           W_t, b_t, W_i, b_i, W_q, b_q, W1, b1, W2, b2):
    del input_x, W_t, b_t
    bsz = input_x_i.shape[0]
    n_t = input_x_i.size
    n_q = input_x_q.size
    nw = 32
    idx_t = input_x_i.reshape(nw, n_t // nw // CH, CH)
    idx_q = input_x_q.reshape(nw, n_q // nw // CH, CH)

    rows_t, rows_q = _sc_gather(table, idx_t, idx_q, n_t, n_q, nw)

    s = 8
    grid = bsz // s
    dense = W1.shape[1]
    ncls = W2.shape[1]
    out = pl.pallas_call(
        functools.partial(_tc_body, s=s),
        grid=(grid,),
        in_specs=[
            pl.BlockSpec((s * 1000, EMB), lambda i: (i, 0)),
            pl.BlockSpec((s * 100, EMB), lambda i: (i, 0)),
            pl.BlockSpec((EMB, EMB), lambda i: (0, 0)),
            pl.BlockSpec((1, EMB), lambda i: (0, 0)),
            pl.BlockSpec((EMB, EMB), lambda i: (0, 0)),
            pl.BlockSpec((1, EMB), lambda i: (0, 0)),
            pl.BlockSpec((2 * EMB, dense), lambda i: (0, 0)),
            pl.BlockSpec((1, dense), lambda i: (0, 0)),
            pl.BlockSpec((dense, ncls), lambda i: (0, 0)),
            pl.BlockSpec((1, ncls), lambda i: (0, 0)),
        ],
        out_specs=pl.BlockSpec((s, ncls), lambda i: (i, 0)),
        out_shape=jax.ShapeDtypeStruct((bsz, ncls), jnp.float32),
    )(rows_t, rows_q, W_i, b_i.reshape(1, EMB), W_q, b_q.reshape(1, EMB),
      W1, b1.reshape(1, dense), W2, b2.reshape(1, ncls))
    return out


# speedup vs baseline: 1.4686x; 1.4686x over previous
import functools

import jax
import jax.numpy as jnp
from jax import lax
from jax.experimental import pallas as pl
from jax.experimental.pallas import tpu as pltpu
from jax.experimental.pallas import tpu_sc as plsc

EMB = 64
CH = 128


def _sc_gather(table, idx_t, idx_q, n_t, n_q, nw):
    info = plsc.get_sparse_core_info()
    nc, ns = info.num_cores, info.num_subcores
    assert nc * ns == nw
    kt = idx_t.shape[1]
    kq = idx_q.shape[1]
    per_t = kt * CH
    per_q = kq * CH

    mesh = plsc.VectorSubcoreMesh(core_axis_name="c", subcore_axis_name="s")

    @functools.partial(
        pl.kernel,
        out_type=(
            jax.ShapeDtypeStruct((n_t, EMB), jnp.float32),
            jax.ShapeDtypeStruct((n_q, EMB), jnp.float32),
        ),
        mesh=mesh,
        compiler_params=pltpu.CompilerParams(use_tc_tiling_on_sc=False),
        scratch_types=[
            pltpu.VMEM((kt, CH), jnp.int32),
            pltpu.VMEM((kq, CH), jnp.int32),
            pltpu.VMEM((CH, EMB), jnp.float32),
            pltpu.VMEM((CH, EMB), jnp.float32),
            pltpu.SemaphoreType.DMA,
            pltpu.SemaphoreType.DMA,
            pltpu.SemaphoreType.DMA,
            pltpu.SemaphoreType.DMA,
        ],
    )
    def k(table_h, idxt_h, idxq_h, out_t_h, out_q_h,
          idxt_v, idxq_v, rows0, rows1, gsem0, gsem1, ssem0, ssem1):
        wid = lax.axis_index("s") * nc + lax.axis_index("c")
        pltpu.sync_copy(idxt_h.at[wid], idxt_v)
        pltpu.sync_copy(idxq_h.at[wid], idxq_v)

        def run(idx_v, out_h, base, kk):
            assert kk >= 2
            rows = (rows0, rows1)
            gsem = (gsem0, gsem1)
            ssem = (ssem0, ssem1)
            pltpu.async_copy(table_h.at[idx_v.at[0]], rows[0], gsem[0])

            def step(jj, _):
                for b in range(2):
                    @pl.when(jj % 2 == b)
                    def _():
                        @pl.when(jj + 1 < kk)
                        def _():
                            @pl.when(jj >= 1)
                            def _():
                                pltpu.make_async_copy(
                                    rows[1 - b], out_h.at[pl.ds(base, CH)],
                                    ssem[1 - b]).wait()

                            pltpu.async_copy(
                                table_h.at[idx_v.at[jj + 1]], rows[1 - b],
                                gsem[1 - b])

                        pltpu.make_async_copy(
                            table_h.at[idx_v.at[jj]], rows[b], gsem[b]).wait()
                        pltpu.async_copy(
                            rows[b], out_h.at[pl.ds(base + jj * CH, CH)],
                            ssem[b])
                return 0

            lax.fori_loop(0, kk, step, 0)
            for jj in (kk - 2, kk - 1):
                pltpu.make_async_copy(
                    rows[jj % 2], out_h.at[pl.ds(base, CH)],
                    ssem[jj % 2]).wait()

        run(idxt_v, out_t_h, wid * per_t, kt)
        run(idxq_v, out_q_h, wid * per_q, kq)

    return k(table, idx_t, idx_q)


def _tc_body(rt_ref, rq_ref, wi_ref, bi_ref, wq_ref, bq_ref,
             w1_ref, b1_ref, w2_ref, b2_ref, out_ref, *, s):
    t = jnp.tanh(rt_ref[...] @ wi_ref[...] + bi_ref[...])
    ts = jnp.sum(t.reshape(s, 500, 2 * EMB), axis=1)
    tm = (ts[:, :EMB] + ts[:, EMB:]) * (1.0 / 1000.0)
    q = jnp.tanh(rq_ref[...] @ wq_ref[...] + bq_ref[...])
    qs = jnp.sum(q.reshape(s, 50, 2 * EMB), axis=1)
    qm = (qs[:, :EMB] + qs[:, EMB:]) * (1.0 / 100.0)
    pool = jnp.concatenate([tm, qm], axis=-1)
    h = jnp.maximum(pool @ w1_ref[...] + b1_ref[...], 0.0)
    out_ref[...] = h @ w2_ref[...] + b2_ref[...]


def kernel(input_x, input_x_i, input_x_q, table,
           W_t, b_t, W_i, b_i, W_q, b_q, W1, b1, W2, b2):
    del input_x, W_t, b_t
    bsz = input_x_i.shape[0]
    n_t = input_x_i.size
    n_q = input_x_q.size
    nw = 32
    idx_t = input_x_i.reshape(nw, n_t // nw // CH, CH)
    idx_q = input_x_q.reshape(nw, n_q // nw // CH, CH)

    rows_t, rows_q = _sc_gather(table, idx_t, idx_q, n_t, n_q, nw)
    rows_t2 = rows_t.reshape(n_t // 2, 2 * EMB)
    rows_q2 = rows_q.reshape(n_q // 2, 2 * EMB)

    zero = jnp.zeros((EMB, EMB), jnp.float32)
    wi_pack = jnp.block([[W_i, zero], [zero, W_i]])
    wq_pack = jnp.block([[W_q, zero], [zero, W_q]])
    bi_pack = jnp.concatenate([b_i, b_i]).reshape(1, 2 * EMB)
    bq_pack = jnp.concatenate([b_q, b_q]).reshape(1, 2 * EMB)

    s = 8
    grid = bsz // s
    dense = W1.shape[1]
    ncls = W2.shape[1]
    out = pl.pallas_call(
        functools.partial(_tc_body, s=s),
        grid=(grid,),
        in_specs=[
            pl.BlockSpec((s * 500, 2 * EMB), lambda i: (i, 0)),
            pl.BlockSpec((s * 50, 2 * EMB), lambda i: (i, 0)),
            pl.BlockSpec((2 * EMB, 2 * EMB), lambda i: (0, 0)),
            pl.BlockSpec((1, 2 * EMB), lambda i: (0, 0)),
            pl.BlockSpec((2 * EMB, 2 * EMB), lambda i: (0, 0)),
            pl.BlockSpec((1, 2 * EMB), lambda i: (0, 0)),
            pl.BlockSpec((2 * EMB, dense), lambda i: (0, 0)),
            pl.BlockSpec((1, dense), lambda i: (0, 0)),
            pl.BlockSpec((dense, ncls), lambda i: (0, 0)),
            pl.BlockSpec((1, ncls), lambda i: (0, 0)),
        ],
        out_specs=pl.BlockSpec((s, ncls), lambda i: (i, 0)),
        out_shape=jax.ShapeDtypeStruct((bsz, ncls), jnp.float32),
    )(rows_t2, rows_q2, wi_pack, bi_pack, wq_pack, bq_pack,
      W1, b1.reshape(1, dense), W2, b2.reshape(1, ncls))
    return out


# speedup vs baseline: 1.5659x; 1.0663x over previous
import functools

import jax
import jax.numpy as jnp
from jax import lax
from jax.experimental import pallas as pl
from jax.experimental.pallas import tpu as pltpu
from jax.experimental.pallas import tpu_sc as plsc

EMB = 64
CH = 128


def _sc_gather(table, idx_t, idx_q, n_t, n_q, nw):
    info = plsc.get_sparse_core_info()
    nc, ns = info.num_cores, info.num_subcores
    assert nc * ns == nw
    kt = idx_t.shape[1]
    kq = idx_q.shape[1]
    per_t = kt * CH
    per_q = kq * CH

    mesh = plsc.VectorSubcoreMesh(core_axis_name="c", subcore_axis_name="s")

    @functools.partial(
        pl.kernel,
        out_type=(
            jax.ShapeDtypeStruct((n_t, EMB), jnp.float32),
            jax.ShapeDtypeStruct((n_q, EMB), jnp.float32),
        ),
        mesh=mesh,
        compiler_params=pltpu.CompilerParams(use_tc_tiling_on_sc=False),
        scratch_types=[
            pltpu.VMEM((kt, CH), jnp.int32),
            pltpu.VMEM((kq, CH), jnp.int32),
            pltpu.VMEM((CH, EMB), jnp.float32),
            pltpu.VMEM((CH, EMB), jnp.float32),
            pltpu.SemaphoreType.DMA,
            pltpu.SemaphoreType.DMA,
            pltpu.SemaphoreType.DMA,
            pltpu.SemaphoreType.DMA,
        ],
    )
    def k(table_h, idxt_h, idxq_h, out_t_h, out_q_h,
          idxt_v, idxq_v, rows0, rows1, gsem0, gsem1, ssem0, ssem1):
        wid = lax.axis_index("s") * nc + lax.axis_index("c")
        pltpu.sync_copy(idxt_h.at[wid], idxt_v)
        pltpu.sync_copy(idxq_h.at[wid], idxq_v)

        def run(idx_v, out_h, base, kk):
            assert kk >= 2
            rows = (rows0, rows1)
            gsem = (gsem0, gsem1)
            ssem = (ssem0, ssem1)
            pltpu.async_copy(table_h.at[idx_v.at[0]], rows[0], gsem[0])

            def step(jj, _):
                for b in range(2):
                    @pl.when(jj % 2 == b)
                    def _():
                        @pl.when(jj + 1 < kk)
                        def _():
                            @pl.when(jj >= 1)
                            def _():
                                pltpu.make_async_copy(
                                    rows[1 - b], out_h.at[pl.ds(base, CH)],
                                    ssem[1 - b]).wait()

                            pltpu.async_copy(
                                table_h.at[idx_v.at[jj + 1]], rows[1 - b],
                                gsem[1 - b])

                        pltpu.make_async_copy(
                            table_h.at[idx_v.at[jj]], rows[b], gsem[b]).wait()
                        pltpu.async_copy(
                            rows[b], out_h.at[pl.ds(base + jj * CH, CH)],
                            ssem[b])
                return 0

            lax.fori_loop(0, kk, step, 0)
            for jj in (kk - 2, kk - 1):
                pltpu.make_async_copy(
                    rows[jj % 2], out_h.at[pl.ds(base, CH)],
                    ssem[jj % 2]).wait()

        run(idxt_v, out_t_h, wid * per_t, kt)
        run(idxq_v, out_q_h, wid * per_q, kq)

    return k(table, idx_t, idx_q)


def _tc_body(rt_ref, rq_ref, wi_ref, bi_ref, wq_ref, bq_ref,
             w1_ref, b1_ref, w2_ref, b2_ref, out_ref, *, sp, mt, mq):
    t = jnp.tanh(rt_ref[...].reshape(mt * sp, 2 * EMB) @ wi_ref[...]
                 + bi_ref[...])
    ts = jnp.sum(t.reshape(mt, sp, 2 * EMB), axis=0) * (1.0 / mt)
    q = jnp.tanh(rq_ref[...].reshape(mq * sp, 2 * EMB) @ wq_ref[...]
                 + bq_ref[...])
    qs = jnp.sum(q.reshape(mq, sp, 2 * EMB), axis=0) * (1.0 / mq)
    pool = jnp.concatenate([ts, qs], axis=-1)
    h = jnp.maximum(pool @ w1_ref[...] + b1_ref[...], 0.0)
    out_ref[...] = h @ w2_ref[...] + b2_ref[...]


def kernel(input_x, input_x_i, input_x_q, table,
           W_t, b_t, W_i, b_i, W_q, b_q, W1, b1, W2, b2):
    del input_x, W_t, b_t
    bsz = input_x_i.shape[0]
    n_t = input_x_i.size
    n_q = input_x_q.size
    mt = n_t // bsz
    mq = n_q // bsz
    nw = 32

    idx_t = input_x_i.transpose(1, 2, 3, 0).reshape(nw, n_t // nw // CH, CH)
    idx_q = input_x_q.transpose(1, 2, 0).reshape(nw, n_q // nw // CH, CH)

    nb = table.shape[0]
    table_lin = jax.lax.optimization_barrier(
        table.reshape(nb // 2, 2 * EMB)).reshape(nb, EMB)

    rows_t, rows_q = _sc_gather(table_lin, idx_t, idx_q, n_t, n_q, nw)
    rows_t3 = rows_t.reshape(mt, bsz // 2, 2 * EMB)
    rows_q3 = rows_q.reshape(mq, bsz // 2, 2 * EMB)

    dense = W1.shape[1]
    ncls = W2.shape[1]
    zero = jnp.zeros((EMB, EMB), jnp.float32)
    wi_pack = jnp.block([[W_i, zero], [zero, W_i]])
    wq_pack = jnp.block([[W_q, zero], [zero, W_q]])
    bi_pack = jnp.concatenate([b_i, b_i]).reshape(1, 2 * EMB)
    bq_pack = jnp.concatenate([b_q, b_q]).reshape(1, 2 * EMB)
    zd = jnp.zeros((EMB, dense), jnp.float32)
    w1_pack = jnp.block([
        [W1[:EMB], zd], [zd, W1[:EMB]], [W1[EMB:], zd], [zd, W1[EMB:]]])
    b1_pack = jnp.concatenate([b1, b1]).reshape(1, 2 * dense)
    zc = jnp.zeros((dense, ncls), jnp.float32)
    w2_pack = jnp.block([[W2, zc], [zc, W2]])
    b2_pack = jnp.concatenate([b2, b2]).reshape(1, 2 * ncls)

    s = 16
    sp = s // 2
    grid = bsz // s
    out = pl.pallas_call(
        functools.partial(_tc_body, sp=sp, mt=mt, mq=mq),
        grid=(grid,),
        in_specs=[
            pl.BlockSpec((mt, sp, 2 * EMB), lambda i: (0, i, 0)),
            pl.BlockSpec((mq, sp, 2 * EMB), lambda i: (0, i, 0)),
            pl.BlockSpec((2 * EMB, 2 * EMB), lambda i: (0, 0)),
            pl.BlockSpec((1, 2 * EMB), lambda i: (0, 0)),
            pl.BlockSpec((2 * EMB, 2 * EMB), lambda i: (0, 0)),
            pl.BlockSpec((1, 2 * EMB), lambda i: (0, 0)),
            pl.BlockSpec((4 * EMB, 2 * dense), lambda i: (0, 0)),
            pl.BlockSpec((1, 2 * dense), lambda i: (0, 0)),
            pl.BlockSpec((2 * dense, 2 * ncls), lambda i: (0, 0)),
            pl.BlockSpec((1, 2 * ncls), lambda i: (0, 0)),
        ],
        out_specs=pl.BlockSpec((sp, 2 * ncls), lambda i: (i, 0)),
        out_shape=jax.ShapeDtypeStruct((bsz // 2, 2 * ncls), jnp.float32),
    )(rows_t3, rows_q3, wi_pack, bi_pack, wq_pack, bq_pack,
      w1_pack, b1_pack, w2_pack, b2_pack)
    return out.reshape(bsz, ncls)
